# baseline (speedup 1.0000x reference)
import functools

import jax
import jax.numpy as jnp
from jax import lax
from jax.experimental import pallas as pl
from jax.experimental.pallas import tpu as pltpu
from jax.experimental.pallas import tpu_sc as plsc

_BLOCK_SIZE = 20
_N_EMBD = 16
_N_EMBD2 = 128
_VOCAB = 100000
_BATCH = 1024

_NC, _NS = 2, 16
_NW = _NC * _NS
_NIDX = _BATCH * _BLOCK_SIZE
_B_PER_W = _NIDX // _NW


def _make_gather():
    mesh = plsc.VectorSubcoreMesh(core_axis_name="c", subcore_axis_name="s")

    @functools.partial(
        pl.kernel,
        mesh=mesh,
        out_type=jax.ShapeDtypeStruct((_NIDX, _N_EMBD), jnp.float32),
        scratch_types=[
            pltpu.VMEM((_B_PER_W,), jnp.int32),
            pltpu.VMEM((_B_PER_W, _N_EMBD), jnp.float32),
            pltpu.SemaphoreType.DMA,
        ],
        compiler_params=pltpu.CompilerParams(use_tc_tiling_on_sc=False),
    )
    def gather_k(table_hbm, idx_hbm, out_hbm, idx_v, rows_v, sem):
        wid = lax.axis_index("s") * _NC + lax.axis_index("c")
        base = wid * _B_PER_W
        pltpu.sync_copy(idx_hbm.at[pl.ds(base, _B_PER_W)], idx_v)
        pltpu.async_copy(table_hbm.at[idx_v], rows_v, sem).wait()
        pltpu.sync_copy(rows_v, out_hbm.at[pl.ds(base, _B_PER_W)])

    return gather_k


_gather_cache = []


def _gather(table, idx):
    if not _gather_cache:
        _gather_cache.append(_make_gather())
    return _gather_cache[0](table, idx)

_NBLK = 2048
_GRID = (_VOCAB + _NBLK - 1) // _NBLK


def _mlp_body(hin_ref, w1_ref, b1_ref, w2_ref, b2_ref, out_ref, h_scr):
    @pl.when(pl.program_id(0) == 0)
    def _():
        h = jnp.dot(hin_ref[...], w1_ref[...], preferred_element_type=jnp.float32)
        h_scr[...] = jnp.tanh(h + b1_ref[...])

    out_ref[...] = (
        jnp.dot(h_scr[...], w2_ref[...], preferred_element_type=jnp.float32)
        + b2_ref[...]
    )


def _mlp(h_in, W1, b1, W2, b2, *, interpret=False):
    return pl.pallas_call(
        _mlp_body,
        grid=(_GRID,),
        in_specs=[
            pl.BlockSpec((_BATCH, _BLOCK_SIZE * _N_EMBD), lambda i: (0, 0)),
            pl.BlockSpec((_BLOCK_SIZE * _N_EMBD, _N_EMBD2), lambda i: (0, 0)),
            pl.BlockSpec((1, _N_EMBD2), lambda i: (0, 0)),
            pl.BlockSpec((_N_EMBD2, _NBLK), lambda i: (0, i)),
            pl.BlockSpec((1, _NBLK), lambda i: (0, i)),
        ],
        out_specs=pl.BlockSpec((_BATCH, _NBLK), lambda i: (0, i)),
        out_shape=jax.ShapeDtypeStruct((_BATCH, _VOCAB), jnp.float32),
        scratch_shapes=[pltpu.VMEM((_BATCH, _N_EMBD2), jnp.float32)],
        compiler_params=pltpu.CompilerParams(
            dimension_semantics=("arbitrary",),
        ),
        interpret=interpret,
    )(h_in, W1, b1, W2, b2)


def kernel(x, wte, W1, b1, W2, b2):
    idx = x.reshape(-1).astype(jnp.int32)
    rows = _gather(wte, idx)
    h_in = rows.reshape(_BATCH, _BLOCK_SIZE * _N_EMBD)
    return _mlp(h_in, W1, b1.reshape(1, _N_EMBD2), W2, b2.reshape(1, _VOCAB))


# speedup vs baseline: 2.8592x; 2.8592x over previous
import functools

import jax
import jax.numpy as jnp
from jax import lax
from jax.experimental import pallas as pl
from jax.experimental.pallas import tpu as pltpu
from jax.experimental.pallas import tpu_sc as plsc

_BLOCK_SIZE = 20
_N_EMBD = 16
_N_EMBD2 = 128
_VOCAB = 100000
_BATCH = 1024

_NC, _NS = 2, 16
_NW = _NC * _NS
_NIDX = _BATCH * _BLOCK_SIZE
_B_PER_W = _NIDX // _NW


def _make_gather():
    mesh = plsc.VectorSubcoreMesh(core_axis_name="c", subcore_axis_name="s")

    @functools.partial(
        pl.kernel,
        mesh=mesh,
        out_type=jax.ShapeDtypeStruct((_NIDX, _N_EMBD), jnp.float32),
        scratch_types=[
            pltpu.VMEM((_B_PER_W,), jnp.int32),
            pltpu.VMEM((_B_PER_W, _N_EMBD), jnp.float32),
            pltpu.SemaphoreType.DMA,
        ],
        compiler_params=pltpu.CompilerParams(use_tc_tiling_on_sc=False),
    )
    def gather_k(table_hbm, idx_hbm, out_hbm, idx_v, rows_v, sem):
        wid = lax.axis_index("s") * _NC + lax.axis_index("c")
        base = wid * _B_PER_W
        pltpu.sync_copy(idx_hbm.at[pl.ds(base, _B_PER_W)], idx_v)
        pltpu.async_copy(table_hbm.at[idx_v], rows_v, sem).wait()
        pltpu.sync_copy(rows_v, out_hbm.at[pl.ds(base, _B_PER_W)])

    return gather_k


_gather_cache = []


def _gather(table, idx):
    if not _gather_cache:
        _gather_cache.append(_make_gather())
    return _gather_cache[0](table, idx)

_NBLK = 2048
_GRID = (_VOCAB + _NBLK - 1) // _NBLK


def _mlp_body(hin_ref, w1_ref, b1_ref, w2t_ref, b2_ref, out_ref, ht_scr):
    @pl.when(pl.program_id(0) == 0)
    def _():
        ht = lax.dot_general(
            w1_ref[...], hin_ref[...],
            (((0,), (1,)), ((), ())),
            preferred_element_type=jnp.float32,
        )
        ht_scr[...] = jnp.tanh(ht + b1_ref[...])

    acc = lax.dot_general(
        w2t_ref[...], ht_scr[...],
        (((1,), (0,)), ((), ())),
        preferred_element_type=jnp.float32,
    )
    out_ref[...] = acc + jnp.transpose(b2_ref[...])


def _mlp_t(h_in, W1, b1c, W2t, b2r, *, interpret=False):
    return pl.pallas_call(
        _mlp_body,
        grid=(_GRID,),
        in_specs=[
            pl.BlockSpec((_BATCH, _BLOCK_SIZE * _N_EMBD), lambda i: (0, 0)),
            pl.BlockSpec((_BLOCK_SIZE * _N_EMBD, _N_EMBD2), lambda i: (0, 0)),
            pl.BlockSpec((_N_EMBD2, 1), lambda i: (0, 0)),
            pl.BlockSpec((_NBLK, _N_EMBD2), lambda i: (i, 0)),
            pl.BlockSpec((1, _NBLK), lambda i: (0, i)),
        ],
        out_specs=pl.BlockSpec((_NBLK, _BATCH), lambda i: (i, 0)),
        out_shape=jax.ShapeDtypeStruct((_VOCAB, _BATCH), jnp.float32),
        scratch_shapes=[pltpu.VMEM((_N_EMBD2, _BATCH), jnp.float32)],
        compiler_params=pltpu.CompilerParams(
            dimension_semantics=("arbitrary",),
        ),
        interpret=interpret,
    )(h_in, W1, b1c, W2t, b2r)


def kernel(x, wte, W1, b1, W2, b2):
    idx = x.reshape(-1).astype(jnp.int32)
    rows = _gather(wte, idx)
    h_in = rows.reshape(_BATCH, _BLOCK_SIZE * _N_EMBD)
    out_t = _mlp_t(
        h_in, W1, b1.reshape(_N_EMBD2, 1), W2.T, b2.reshape(1, _VOCAB)
    )
    return out_t.T


# speedup vs baseline: 3.2107x; 1.1229x over previous
import functools

import jax
import jax.numpy as jnp
from jax import lax
from jax.experimental import pallas as pl
from jax.experimental.pallas import tpu as pltpu
from jax.experimental.pallas import tpu_sc as plsc

_BLOCK_SIZE = 20
_N_EMBD = 16
_N_EMBD2 = 128
_VOCAB = 100000
_BATCH = 1024

_NC, _NS = 2, 16
_NW = _NC * _NS
_NELEM = _BATCH * _BLOCK_SIZE * _N_EMBD
_E_PER_W = _NELEM // _NW


def _make_gather():
    mesh = plsc.VectorSubcoreMesh(core_axis_name="c", subcore_axis_name="s")

    @functools.partial(
        pl.kernel,
        mesh=mesh,
        out_type=jax.ShapeDtypeStruct((_NELEM,), jnp.float32),
        scratch_types=[
            pltpu.VMEM((_E_PER_W,), jnp.int32),
            pltpu.VMEM((_E_PER_W,), jnp.float32),
            pltpu.SemaphoreType.DMA,
        ],
        compiler_params=pltpu.CompilerParams(use_tc_tiling_on_sc=False),
    )
    def gather_k(table_hbm, idx_hbm, out_hbm, idx_v, elts_v, sem):
        wid = lax.axis_index("s") * _NC + lax.axis_index("c")
        base = wid * _E_PER_W
        pltpu.sync_copy(idx_hbm.at[pl.ds(base, _E_PER_W)], idx_v)
        pltpu.async_copy(table_hbm.at[idx_v], elts_v, sem).wait()
        pltpu.sync_copy(elts_v, out_hbm.at[pl.ds(base, _E_PER_W)])

    return gather_k


_gather_cache = []


def _gather(table_flat, idx):
    if not _gather_cache:
        _gather_cache.append(_make_gather())
    return _gather_cache[0](table_flat, idx)

_NBLK = 2048
_GRID = (_VOCAB + _NBLK - 1) // _NBLK


def _mlp_body(hint_ref, w1_ref, b1_ref, w2t_ref, b2_ref, out_ref, ht_scr):
    @pl.when(pl.program_id(0) == 0)
    def _():
        ht = lax.dot_general(
            w1_ref[...], hint_ref[...],
            (((0,), (0,)), ((), ())),
            preferred_element_type=jnp.float32,
        )
        ht_scr[...] = jnp.tanh(ht + b1_ref[...])

    acc = lax.dot_general(
        w2t_ref[...], ht_scr[...],
        (((1,), (0,)), ((), ())),
        preferred_element_type=jnp.float32,
    )
    out_ref[...] = acc + jnp.transpose(b2_ref[...])


def _mlp_t(h_in_t, W1, b1c, W2t, b2r, *, interpret=False):
    return pl.pallas_call(
        _mlp_body,
        grid=(_GRID,),
        in_specs=[
            pl.BlockSpec((_BLOCK_SIZE * _N_EMBD, _BATCH), lambda i: (0, 0)),
            pl.BlockSpec((_BLOCK_SIZE * _N_EMBD, _N_EMBD2), lambda i: (0, 0)),
            pl.BlockSpec((_N_EMBD2, 1), lambda i: (0, 0)),
            pl.BlockSpec((_NBLK, _N_EMBD2), lambda i: (i, 0)),
            pl.BlockSpec((1, _NBLK), lambda i: (0, i)),
        ],
        out_specs=pl.BlockSpec((_NBLK, _BATCH), lambda i: (i, 0)),
        out_shape=jax.ShapeDtypeStruct((_VOCAB, _BATCH), jnp.float32),
        scratch_shapes=[pltpu.VMEM((_N_EMBD2, _BATCH), jnp.float32)],
        compiler_params=pltpu.CompilerParams(
            dimension_semantics=("arbitrary",),
        ),
        interpret=interpret,
    )(h_in_t, W1, b1c, W2t, b2r)


def kernel(x, wte, W1, b1, W2, b2):
    xt = x.T.astype(jnp.int32)
    d_off = (jnp.arange(_N_EMBD, dtype=jnp.int32) * _VOCAB)
    idx2 = (xt[:, None, :] + d_off[None, :, None]).reshape(_NELEM)
    table = wte.T.reshape(_N_EMBD * _VOCAB)
    elts = _gather(table, idx2)
    h_in_t = elts.reshape(_BLOCK_SIZE * _N_EMBD, _BATCH)
    out_t = _mlp_t(
        h_in_t, W1, b1.reshape(_N_EMBD2, 1), W2.T, b2.reshape(1, _VOCAB)
    )
    return out_t.T


# speedup vs baseline: 3.2744x; 1.0198x over previous
import functools

import jax
import jax.numpy as jnp
from jax import lax
from jax.experimental import pallas as pl
from jax.experimental.pallas import tpu as pltpu
from jax.experimental.pallas import tpu_sc as plsc

_BLOCK_SIZE = 20
_N_EMBD = 16
_N_EMBD2 = 128
_VOCAB = 100000
_BATCH = 1024

_NC, _NS = 2, 16
_NW = _NC * _NS
_NELEM = _BATCH * _BLOCK_SIZE * _N_EMBD
_E_PER_W = _NELEM // _NW


def _make_gather():
    mesh = plsc.VectorSubcoreMesh(core_axis_name="c", subcore_axis_name="s")

    @functools.partial(
        pl.kernel,
        mesh=mesh,
        out_type=jax.ShapeDtypeStruct((_NELEM,), jnp.float32),
        scratch_types=[
            pltpu.VMEM((_E_PER_W,), jnp.int32),
            pltpu.VMEM((_E_PER_W,), jnp.float32),
            pltpu.SemaphoreType.DMA,
        ],
        compiler_params=pltpu.CompilerParams(use_tc_tiling_on_sc=False),
    )
    def gather_k(table_hbm, idx_hbm, out_hbm, idx_v, elts_v, sem):
        wid = lax.axis_index("s") * _NC + lax.axis_index("c")
        base = wid * _E_PER_W
        pltpu.sync_copy(idx_hbm.at[pl.ds(base, _E_PER_W)], idx_v)
        pltpu.async_copy(table_hbm.at[idx_v], elts_v, sem).wait()
        pltpu.sync_copy(elts_v, out_hbm.at[pl.ds(base, _E_PER_W)])

    return gather_k


_gather_cache = []


def _gather(table_flat, idx):
    if not _gather_cache:
        _gather_cache.append(_make_gather())
    return _gather_cache[0](table_flat, idx)

_NBLK = 4096
_GRID = (_VOCAB + _NBLK - 1) // _NBLK


def _mlp_body(hint_ref, w1_ref, b1_ref, w2t_ref, b2_ref, out_ref, ht_scr):
    @pl.when(pl.program_id(0) == 0)
    def _():
        ht = lax.dot_general(
            w1_ref[...], hint_ref[...],
            (((0,), (0,)), ((), ())),
            preferred_element_type=jnp.float32,
        )
        ht_scr[...] = jnp.tanh(ht + b1_ref[...])

    acc = lax.dot_general(
        w2t_ref[...], ht_scr[...],
        (((1,), (0,)), ((), ())),
        preferred_element_type=jnp.float32,
    )
    out_ref[...] = acc + jnp.transpose(b2_ref[...])


def _mlp_t(h_in_t, W1, b1c, W2t, b2r, *, interpret=False):
    return pl.pallas_call(
        _mlp_body,
        grid=(_GRID,),
        in_specs=[
            pl.BlockSpec((_BLOCK_SIZE * _N_EMBD, _BATCH), lambda i: (0, 0)),
            pl.BlockSpec((_BLOCK_SIZE * _N_EMBD, _N_EMBD2), lambda i: (0, 0)),
            pl.BlockSpec((_N_EMBD2, 1), lambda i: (0, 0)),
            pl.BlockSpec((_NBLK, _N_EMBD2), lambda i: (i, 0)),
            pl.BlockSpec((1, _NBLK), lambda i: (0, i)),
        ],
        out_specs=pl.BlockSpec((_NBLK, _BATCH), lambda i: (i, 0)),
        out_shape=jax.ShapeDtypeStruct((_VOCAB, _BATCH), jnp.float32),
        scratch_shapes=[pltpu.VMEM((_N_EMBD2, _BATCH), jnp.float32)],
        compiler_params=pltpu.CompilerParams(
            dimension_semantics=("arbitrary",),
        ),
        interpret=interpret,
    )(h_in_t, W1, b1c, W2t, b2r)


def kernel(x, wte, W1, b1, W2, b2):
    xt = x.T.astype(jnp.int32)
    d_off = (jnp.arange(_N_EMBD, dtype=jnp.int32) * _VOCAB)
    idx2 = (xt[:, None, :] + d_off[None, :, None]).reshape(_NELEM)
    table = wte.T.reshape(_N_EMBD * _VOCAB)
    elts = _gather(table, idx2)
    h_in_t = elts.reshape(_BLOCK_SIZE * _N_EMBD, _BATCH)
    out_t = _mlp_t(
        h_in_t, W1, b1.reshape(_N_EMBD2, 1), W2.T, b2.reshape(1, _VOCAB)
    )
    return out_t.T
